# Initial kernel scaffold; baseline (speedup 1.0000x reference)
import jax
import jax.numpy as jnp
from jax.experimental import pallas as pl


def kernel(x, edge_index, att, Wz, bz, Wlz, blz, Wr, br, Wlr, blr, Wh, bh, Wlh, blh, W1, b1, W2, b2):
    raise NotImplementedError("write your pallas kernel here")


# speedup vs baseline: 21.3060x; 21.3060x over previous
import jax
import jax.numpy as jnp
from jax.experimental import pallas as pl

_N = 21
_OC = 32
_P = 12
_F = 128
_E = 400


def _prep_kernel(ei_ref, wz_ref, wlz_ref, wh_ref, wlh_ref,
                 bz_ref, blz_ref, bh_ref, blh_ref,
                 a_ref, wc_ref, cvec_ref):
    ei = ei_ref[:]
    s = ei[0:1, :]
    d = ei[1:2, :]
    nodes = jax.lax.broadcasted_iota(jnp.int32, (_N, _E), 0)
    os_t = (s == nodes).astype(jnp.float32)
    od_t = (d == nodes).astype(jnp.float32)
    deg = jnp.sum(od_t, axis=1, keepdims=True) + 1.0
    dinv = jax.lax.rsqrt(deg)
    dn = (((0,), (0,)), ((), ()))
    dinv_s = jax.lax.dot_general(dinv, os_t, dn,
                                 preferred_element_type=jnp.float32)
    dinv_d = jax.lax.dot_general(dinv, od_t, dn,
                                 preferred_element_type=jnp.float32)
    norm = dinv_s * dinv_d
    dn1 = (((1,), (1,)), ((), ()))
    a = jax.lax.dot_general(od_t * norm, os_t, dn1,
                            preferred_element_type=jnp.float32)
    r = jax.lax.broadcasted_iota(jnp.int32, (_N, _N), 0)
    c = jax.lax.broadcasted_iota(jnp.int32, (_N, _N), 1)
    a_ref[:] = a + jnp.where(r == c, dinv * dinv, 0.0)

    wlz_t = wlz_ref[0:_OC, :]
    wlh_t = wlh_ref[0:_OC, :]
    wcz = jnp.dot(wz_ref[:], wlz_t, preferred_element_type=jnp.float32)
    wch = jnp.dot(wh_ref[:], wlh_t, preferred_element_type=jnp.float32)
    wc_ref[:] = jnp.concatenate([wcz, wch], axis=1)
    cz = jnp.dot(bz_ref[:], wlz_t, preferred_element_type=jnp.float32) + blz_ref[:]
    ch = jnp.dot(bh_ref[:], wlh_t, preferred_element_type=jnp.float32) + blh_ref[:]
    cvec_ref[:] = jnp.concatenate([cz, ch], axis=1)


def _main_kernel(x_ref, a_ref, wc_ref, cvec_ref, att_ref,
                 w1_ref, b1_ref, w2_ref, b2_ref, out_ref):
    att = att_ref[:]
    m = jnp.max(att, axis=1, keepdims=True)
    e = jnp.exp(att - m)
    pr = e / jnp.sum(e, axis=1, keepdims=True)

    a = a_ref[:]
    wc = wc_ref[:]
    cvec = cvec_ref[:]
    acc = jnp.zeros((_N, _OC), dtype=jnp.float32)
    for p in range(_P):
        xp = x_ref[0, p]
        y = jnp.dot(xp, wc, preferred_element_type=jnp.float32)
        ay = jnp.dot(a, y, preferred_element_type=jnp.float32) + cvec
        z = jax.nn.sigmoid(ay[:, 0:_OC])
        t = jnp.tanh(ay[:, _OC:2 * _OC])
        acc = acc + pr[0:1, p:p + 1] * (1.0 - z) * t

    hr = jnp.maximum(acc, 0.0)
    h1 = jnp.dot(hr, w1_ref[:], preferred_element_type=jnp.float32) + b1_ref[:]
    dn = (((0,), (0,)), ((), ()))
    out = jax.lax.dot_general(w2_ref[:], h1, dn,
                              preferred_element_type=jnp.float32)
    out_ref[0] = out + b2_ref[:]


def kernel(x, edge_index, att, Wz, bz, Wlz, blz, Wr, br, Wlr, blr,
           Wh, bh, Wlh, blh, W1, b1, W2, b2):
    B = x.shape[0]
    xt = jnp.transpose(x, (0, 3, 1, 2))

    a, wc, cvec = pl.pallas_call(
        _prep_kernel,
        out_shape=[
            jax.ShapeDtypeStruct((_N, _N), jnp.float32),
            jax.ShapeDtypeStruct((_F, 2 * _OC), jnp.float32),
            jax.ShapeDtypeStruct((1, 2 * _OC), jnp.float32),
        ],
    )(edge_index, Wz, Wlz, Wh, Wlh,
      bz.reshape(1, _OC), blz.reshape(1, _OC),
      bh.reshape(1, _OC), blh.reshape(1, _OC))

    full = lambda shape: pl.BlockSpec(shape, lambda b: (0,) * len(shape))
    out = pl.pallas_call(
        _main_kernel,
        grid=(B,),
        in_specs=[
            pl.BlockSpec((1, _P, _N, _F), lambda b: (b, 0, 0, 0)),
            full((_N, _N)),
            full((_F, 2 * _OC)),
            full((1, 2 * _OC)),
            full((1, _P)),
            full((_OC, _P)),
            full((1, _P)),
            full((_N, 6)),
            full((6, 1)),
        ],
        out_specs=pl.BlockSpec((1, 6, _P), lambda b: (b, 0, 0)),
        out_shape=jax.ShapeDtypeStruct((B, 6, _P), jnp.float32),
    )(xt, a, wc, cvec, att.reshape(1, _P),
      W1, b1.reshape(1, _P), W2, b2.reshape(6, 1))
    return out


# speedup vs baseline: 53.8688x; 2.5283x over previous
import numpy as np
import jax
import jax.numpy as jnp
from jax.experimental import pallas as pl

_N = 21
_OC = 32
_P = 12
_F = 128
_E = 400
_KB = 4
_R = _KB * _P * _N
_R2 = _KB * _N
_C2 = _KB * 6

_r = np.arange(_R)
_T1 = (_r[:, None] % _N == np.arange(_N)[None, :]).astype(np.float32)
_GID_C = (_r[:, None] // _N).astype(np.float32)
_r2 = np.arange(_R2)
_EP = ((_r[None, :] % (_P * _N)) // _N == np.arange(_P)[:, None]).astype(np.float32)
_QMOD = (_r[None, :] % _N).astype(np.float32)
_RMOD = (_r2[:, None] % _N).astype(np.float32)
_QGRP = (_r[None, :] // (_P * _N)).astype(np.float32)
_RGRP = (_r2[:, None] // _N).astype(np.float32)
_T1B = (_r2[:, None] % _N == np.arange(_N)[None, :]).astype(np.float32)
_c2 = np.arange(_C2)
_T2C = (np.arange(6)[:, None] == _c2[None, :] % 6).astype(np.float32)
_CGRP = (_c2[None, :] // 6).astype(np.float32)


def _prep_kernel(ei_ref, att_ref, wz_ref, wlz_ref, wh_ref, wlh_ref,
                 bz_ref, blz_ref, bh_ref, blh_ref, w2_ref,
                 t1_ref, gidc_ref, ep_ref, qmod_ref, rmod_ref,
                 qgrp_ref, rgrp_ref, t1b_ref, t2c_ref, cgrp_ref,
                 abig_ref, wc_ref, cvec_ref, psel_ref, w2big_ref):
    f32 = jnp.float32
    dn0 = (((0,), (0,)), ((), ()))
    dn1 = (((1,), (1,)), ((), ()))

    ei = ei_ref[:]
    s = ei[0:1, :]
    d = ei[1:2, :]
    nodes = jax.lax.broadcasted_iota(jnp.int32, (_N, _E), 0)
    os_t = (s == nodes).astype(f32)
    od_t = (d == nodes).astype(f32)
    deg = jnp.sum(od_t, axis=1, keepdims=True) + 1.0
    dinv = jax.lax.rsqrt(deg)
    dinv_s = jax.lax.dot_general(dinv, os_t, dn0, preferred_element_type=f32)
    dinv_d = jax.lax.dot_general(dinv, od_t, dn0, preferred_element_type=f32)
    norm = dinv_s * dinv_d
    a = jax.lax.dot_general(od_t * norm, os_t, dn1, preferred_element_type=f32)
    rr = jax.lax.broadcasted_iota(jnp.int32, (_N, _N), 0)
    cc = jax.lax.broadcasted_iota(jnp.int32, (_N, _N), 1)
    a = a + jnp.where(rr == cc, dinv * dinv, 0.0)

    t1 = t1_ref[:]
    tile = jnp.dot(jnp.dot(t1, a, preferred_element_type=f32),
                   jnp.transpose(t1), preferred_element_type=f32)
    gidc = gidc_ref[:]
    blk = (gidc == jnp.transpose(gidc)).astype(f32)
    abig_ref[:] = tile * blk

    wlz_t = wlz_ref[0:_OC, :]
    wlh_t = wlh_ref[0:_OC, :]
    wcz = jnp.dot(wz_ref[:], wlz_t, preferred_element_type=f32)
    wch = jnp.dot(wh_ref[:], wlh_t, preferred_element_type=f32)
    wc_ref[:] = jnp.concatenate([wcz, wch], axis=1)
    cz = jnp.dot(bz_ref[:], wlz_t, preferred_element_type=f32) + blz_ref[:]
    ch = jnp.dot(bh_ref[:], wlh_t, preferred_element_type=f32) + blh_ref[:]
    cvec_ref[:] = jnp.concatenate([cz, ch], axis=1)

    att = att_ref[:]
    m = jnp.max(att, axis=1, keepdims=True)
    e = jnp.exp(att - m)
    pr = e / jnp.sum(e, axis=1, keepdims=True)
    prexp = jnp.dot(pr, ep_ref[:], preferred_element_type=f32)
    maska = (qmod_ref[:] == rmod_ref[:]).astype(f32)
    maskb = (qgrp_ref[:] == rgrp_ref[:]).astype(f32)
    psel_ref[:] = maska * maskb * prexp

    t2 = jnp.dot(jnp.dot(t1b_ref[:], w2_ref[:], preferred_element_type=f32),
                 t2c_ref[:], preferred_element_type=f32)
    mask2 = (rgrp_ref[:] == cgrp_ref[:]).astype(f32)
    w2big_ref[:] = t2 * mask2


def _main_kernel(x_ref, abig_ref, wc_ref, cvec_ref, psel_ref,
                 w1_ref, b1_ref, w2big_ref, b2big_ref, out_ref):
    f32 = jnp.float32
    y = jnp.dot(x_ref[:], wc_ref[:], preferred_element_type=f32)
    ay = jnp.dot(abig_ref[:], y, preferred_element_type=f32) + cvec_ref[:]
    g = (1.0 - jax.nn.sigmoid(ay[:, 0:_OC])) * jnp.tanh(ay[:, _OC:2 * _OC])
    res = jnp.dot(psel_ref[:], g, preferred_element_type=f32)
    hr = jnp.maximum(res, 0.0)
    h1 = jnp.dot(hr, w1_ref[:], preferred_element_type=f32) + b1_ref[:]
    dn0 = (((0,), (0,)), ((), ()))
    o = jax.lax.dot_general(w2big_ref[:], h1, dn0,
                            preferred_element_type=f32) + b2big_ref[:]
    for i in range(_KB):
        out_ref[i] = o[6 * i:6 * (i + 1), :]


def kernel(x, edge_index, att, Wz, bz, Wlz, blz, Wr, br, Wlr, blr,
           Wh, bh, Wlh, blh, W1, b1, W2, b2):
    B = x.shape[0]
    xt = jnp.transpose(x, (0, 3, 1, 2)).reshape(B * _P * _N, _F)

    abig, wc, cvec, psel, w2big = pl.pallas_call(
        _prep_kernel,
        out_shape=[
            jax.ShapeDtypeStruct((_R, _R), jnp.float32),
            jax.ShapeDtypeStruct((_F, 2 * _OC), jnp.float32),
            jax.ShapeDtypeStruct((1, 2 * _OC), jnp.float32),
            jax.ShapeDtypeStruct((_R2, _R), jnp.float32),
            jax.ShapeDtypeStruct((_R2, _C2), jnp.float32),
        ],
    )(edge_index, att.reshape(1, _P), Wz, Wlz, Wh, Wlh,
      bz.reshape(1, _OC), blz.reshape(1, _OC),
      bh.reshape(1, _OC), blh.reshape(1, _OC), W2,
      jnp.asarray(_T1), jnp.asarray(_GID_C), jnp.asarray(_EP),
      jnp.asarray(_QMOD), jnp.asarray(_RMOD), jnp.asarray(_QGRP),
      jnp.asarray(_RGRP), jnp.asarray(_T1B), jnp.asarray(_T2C),
      jnp.asarray(_CGRP))

    full = lambda shape: pl.BlockSpec(shape, lambda b: (0,) * len(shape))
    out = pl.pallas_call(
        _main_kernel,
        grid=(B // _KB,),
        in_specs=[
            pl.BlockSpec((_R, _F), lambda b: (b, 0)),
            full((_R, _R)),
            full((_F, 2 * _OC)),
            full((1, 2 * _OC)),
            full((_R2, _R)),
            full((_OC, _P)),
            full((1, _P)),
            full((_R2, _C2)),
            full((_C2, 1)),
        ],
        out_specs=pl.BlockSpec((_KB, 6, _P), lambda b: (b, 0, 0)),
        out_shape=jax.ShapeDtypeStruct((B, 6, _P), jnp.float32),
    )(xt, abig, wc, cvec, psel,
      W1, b1.reshape(1, _P), w2big, jnp.tile(b2.reshape(6, 1), (_KB, 1)))
    return out


# speedup vs baseline: 61.8523x; 1.1482x over previous
import numpy as np
import jax
import jax.numpy as jnp
from jax.experimental import pallas as pl
from jax.experimental.pallas import tpu as pltpu

_N = 21
_OC = 32
_P = 12
_F = 128
_E = 400
_KB = 4
_R = _KB * _P * _N
_R2 = _KB * _N
_C2 = _KB * 6

_r = np.arange(_R)
_T1 = (_r[:, None] % _N == np.arange(_N)[None, :]).astype(np.float32)
_GID_C = (_r[:, None] // _N).astype(np.float32)
_r2 = np.arange(_R2)
_EP = ((_r[None, :] % (_P * _N)) // _N == np.arange(_P)[:, None]).astype(np.float32)
_QMOD = (_r[None, :] % _N).astype(np.float32)
_RMOD = (_r2[:, None] % _N).astype(np.float32)
_QGRP = (_r[None, :] // (_P * _N)).astype(np.float32)
_RGRP = (_r2[:, None] // _N).astype(np.float32)
_T1B = (_r2[:, None] % _N == np.arange(_N)[None, :]).astype(np.float32)
_c2 = np.arange(_C2)
_T2C = (np.arange(6)[:, None] == _c2[None, :] % 6).astype(np.float32)
_CGRP = (_c2[None, :] // 6).astype(np.float32)


def _fused_kernel(x_ref, ei_ref, att_ref, wz_ref, wlz_ref, wh_ref, wlh_ref,
                  bz_ref, blz_ref, bh_ref, blh_ref, w2_ref,
                  w1_ref, b1_ref, b2big_ref,
                  t1_ref, gidc_ref, ep_ref, qmod_ref, rmod_ref,
                  qgrp_ref, rgrp_ref, t1b_ref, t2c_ref, cgrp_ref,
                  out_ref,
                  abig_s, wc_s, cvec_s, psel_s, w2big_s):
    f32 = jnp.float32
    dn0 = (((0,), (0,)), ((), ()))
    dn1 = (((1,), (1,)), ((), ()))

    @pl.when(pl.program_id(0) == 0)
    def _prep():
        ei = ei_ref[:]
        s = ei[0:1, :]
        d = ei[1:2, :]
        nodes = jax.lax.broadcasted_iota(jnp.int32, (_N, _E), 0)
        os_t = (s == nodes).astype(f32)
        od_t = (d == nodes).astype(f32)
        deg = jnp.sum(od_t, axis=1, keepdims=True) + 1.0
        dinv = jax.lax.rsqrt(deg)
        dinv_s = jax.lax.dot_general(dinv, os_t, dn0, preferred_element_type=f32)
        dinv_d = jax.lax.dot_general(dinv, od_t, dn0, preferred_element_type=f32)
        norm = dinv_s * dinv_d
        a = jax.lax.dot_general(od_t * norm, os_t, dn1, preferred_element_type=f32)
        rr = jax.lax.broadcasted_iota(jnp.int32, (_N, _N), 0)
        cc = jax.lax.broadcasted_iota(jnp.int32, (_N, _N), 1)
        a = a + jnp.where(rr == cc, dinv * dinv, 0.0)

        t1 = t1_ref[:]
        tile = jnp.dot(jnp.dot(t1, a, preferred_element_type=f32),
                       jnp.transpose(t1), preferred_element_type=f32)
        gidc = gidc_ref[:]
        blk = (gidc == jnp.transpose(gidc)).astype(f32)
        abig_s[:] = tile * blk

        wlz_t = wlz_ref[0:_OC, :]
        wlh_t = wlh_ref[0:_OC, :]
        wcz = jnp.dot(wz_ref[:], wlz_t, preferred_element_type=f32)
        wch = jnp.dot(wh_ref[:], wlh_t, preferred_element_type=f32)
        wc_s[:] = jnp.concatenate([wcz, wch], axis=1)
        cz = jnp.dot(bz_ref[:], wlz_t, preferred_element_type=f32) + blz_ref[:]
        ch = jnp.dot(bh_ref[:], wlh_t, preferred_element_type=f32) + blh_ref[:]
        cvec_s[:] = jnp.concatenate([cz, ch], axis=1)

        att = att_ref[:]
        m = jnp.max(att, axis=1, keepdims=True)
        e = jnp.exp(att - m)
        pr = e / jnp.sum(e, axis=1, keepdims=True)
        prexp = jnp.dot(pr, ep_ref[:], preferred_element_type=f32)
        maska = (qmod_ref[:] == rmod_ref[:]).astype(f32)
        maskb = (qgrp_ref[:] == rgrp_ref[:]).astype(f32)
        psel_s[:] = maska * maskb * prexp

        t2 = jnp.dot(jnp.dot(t1b_ref[:], w2_ref[:], preferred_element_type=f32),
                     t2c_ref[:], preferred_element_type=f32)
        mask2 = (rgrp_ref[:] == cgrp_ref[:]).astype(f32)
        w2big_s[:] = t2 * mask2

    y = jnp.dot(x_ref[:], wc_s[:], preferred_element_type=f32)
    ay = jnp.dot(abig_s[:], y, preferred_element_type=f32) + cvec_s[:]
    g = (1.0 - jax.nn.sigmoid(ay[:, 0:_OC])) * jnp.tanh(ay[:, _OC:2 * _OC])
    res = jnp.dot(psel_s[:], g, preferred_element_type=f32)
    hr = jnp.maximum(res, 0.0)
    h1 = jnp.dot(hr, w1_ref[:], preferred_element_type=f32) + b1_ref[:]
    o = jax.lax.dot_general(w2big_s[:], h1, dn0,
                            preferred_element_type=f32) + b2big_ref[:]
    for i in range(_KB):
        out_ref[i] = o[6 * i:6 * (i + 1), :]


def kernel(x, edge_index, att, Wz, bz, Wlz, blz, Wr, br, Wlr, blr,
           Wh, bh, Wlh, blh, W1, b1, W2, b2):
    B = x.shape[0]
    xt = jnp.transpose(x, (0, 3, 1, 2)).reshape(B * _P * _N, _F)

    full = lambda shape: pl.BlockSpec(shape, lambda b: (0,) * len(shape))
    out = pl.pallas_call(
        _fused_kernel,
        grid=(B // _KB,),
        in_specs=[
            pl.BlockSpec((_R, _F), lambda b: (b, 0)),
            full((2, _E)),
            full((1, _P)),
            full((_F, _OC)),
            full((2 * _OC, _OC)),
            full((_F, _OC)),
            full((2 * _OC, _OC)),
            full((1, _OC)),
            full((1, _OC)),
            full((1, _OC)),
            full((1, _OC)),
            full((_N, 6)),
            full((_OC, _P)),
            full((1, _P)),
            full((_C2, 1)),
            full((_R, _N)),
            full((_R, 1)),
            full((_P, _R)),
            full((1, _R)),
            full((_R2, 1)),
            full((1, _R)),
            full((_R2, 1)),
            full((_R2, _N)),
            full((6, _C2)),
            full((1, _C2)),
        ],
        out_specs=pl.BlockSpec((_KB, 6, _P), lambda b: (b, 0, 0)),
        out_shape=jax.ShapeDtypeStruct((B, 6, _P), jnp.float32),
        scratch_shapes=[
            pltpu.VMEM((_R, _R), jnp.float32),
            pltpu.VMEM((_F, 2 * _OC), jnp.float32),
            pltpu.VMEM((1, 2 * _OC), jnp.float32),
            pltpu.VMEM((_R2, _R), jnp.float32),
            pltpu.VMEM((_R2, _C2), jnp.float32),
        ],
    )(xt, edge_index, att.reshape(1, _P), Wz, Wlz, Wh, Wlh,
      bz.reshape(1, _OC), blz.reshape(1, _OC),
      bh.reshape(1, _OC), blh.reshape(1, _OC), W2,
      W1, b1.reshape(1, _P), jnp.tile(b2.reshape(6, 1), (_KB, 1)),
      jnp.asarray(_T1), jnp.asarray(_GID_C), jnp.asarray(_EP),
      jnp.asarray(_QMOD), jnp.asarray(_RMOD), jnp.asarray(_QGRP),
      jnp.asarray(_RGRP), jnp.asarray(_T1B), jnp.asarray(_T2C),
      jnp.asarray(_CGRP))
    return out
